# Initial kernel scaffold; baseline (speedup 1.0000x reference)
import jax
import jax.numpy as jnp
from jax.experimental import pallas as pl


def kernel(x, lengths):
    raise NotImplementedError("write your pallas kernel here")


# speedup vs baseline: 1.6692x; 1.6692x over previous
import functools

import jax
import jax.numpy as jnp
from jax import lax
from jax.experimental import pallas as pl
from jax.experimental.pallas import tpu as pltpu
from jax.experimental.pallas import tpu_sc as plsc

N = 32768
D = 512
B = 16
L = 16
NW = 32
COLS = D // NW
CH = 128


def _body(x_hbm, len_hbm, start_hbm, out_hbm, len_v, start_v, buf, res, sem):
    wid = lax.axis_index("s") * 2 + lax.axis_index("c")
    colbase = wid * COLS

    pltpu.sync_copy(len_hbm, len_v)
    pltpu.sync_copy(start_hbm, start_v)
    lens = len_v[...]
    starts = start_v[...]

    for b in range(B):
        n_b = lens[b]
        s_b = starts[b]
        nch = (n_b + CH - 1) // CH
        hi = s_b + n_b

        def chunk_body(c, acc, s_b=s_b, hi=hi):
            row0 = s_b + c * CH
            p = jnp.minimum(row0, N - CH)
            pltpu.sync_copy(
                x_hbm.at[pl.ds(p, CH), pl.ds(colbase, COLS)], buf)

            def row_body(r, a, p=p, row0=row0, hi=hi):
                g = p + r
                ok = jnp.logical_and(g >= row0, g < hi)
                return a + jnp.where(ok, buf[r], jnp.float32(0.0))

            return lax.fori_loop(0, CH, row_body, acc)

        acc = lax.fori_loop(0, nch, chunk_body, jnp.zeros((L,), jnp.float32))
        res[b, :] = acc / n_b.astype(jnp.float32)

    pltpu.sync_copy(res, out_hbm.at[:, pl.ds(colbase, COLS)])


def kernel(x, lengths):
    mesh = plsc.VectorSubcoreMesh(core_axis_name="c", subcore_axis_name="s")
    f = functools.partial(
        pl.kernel,
        mesh=mesh,
        out_type=jax.ShapeDtypeStruct((B, D), jnp.float32),
        scratch_types=[
            pltpu.VMEM((B,), jnp.int32),
            pltpu.VMEM((B,), jnp.int32),
            pltpu.VMEM((CH, L), jnp.float32),
            pltpu.VMEM((B, L), jnp.float32),
            pltpu.SemaphoreType.DMA,
        ],
        compiler_params=pltpu.CompilerParams(use_tc_tiling_on_sc=False),
    )(_body)
    starts = jnp.cumsum(lengths) - lengths
    return f(x, lengths, starts)


# speedup vs baseline: 4.1236x; 2.4704x over previous
import functools

import jax
import jax.numpy as jnp
from jax import lax
from jax.experimental import pallas as pl
from jax.experimental.pallas import tpu as pltpu
from jax.experimental.pallas import tpu_sc as plsc

N = 32768
D = 512
B = 16
L = 16
NW = 32
COLS = D // NW
CH = 256
NBUF = 4


def _body(x_hbm, cum_hbm, out_hbm, cum_v, buf, res, sem):
    wid = lax.axis_index("s") * 2 + lax.axis_index("c")
    colbase = wid * COLS

    pltpu.sync_copy(cum_hbm, cum_v)
    cumv = cum_v[...]
    scs = [cumv[j] for j in range(B)]
    total = scs[B - 1]
    nch = (total + CH - 1) // CH

    zero = jnp.zeros((L,), jnp.float32)
    for b in range(B):
        res[b, :] = zero

    def dcopy(c, slot):
        return pltpu.make_async_copy(
            x_hbm.at[pl.ds(c * CH, CH), pl.ds(colbase, COLS)],
            buf.at[pl.ds(slot * CH, CH)],
            sem.at[slot],
        )

    for k in range(NBUF):
        @pl.when(k < nch)
        def _():
            dcopy(jnp.int32(k), jnp.int32(k)).start()

    def chunk_step(c, carry):
        slot = lax.rem(c, NBUF)
        dcopy(c, slot).wait()
        g0 = c * CH
        g1 = jnp.minimum(g0 + CH, total)
        base = slot * CH - g0

        def accum(rl, rh):
            n8 = lax.div(rh - rl, 8)
            mid = rl + n8 * 8

            def blk(k, a):
                r = rl + k * 8
                a0 = a[0] + buf[r] + buf[r + 4]
                a1 = a[1] + buf[r + 1] + buf[r + 5]
                a2 = a[2] + buf[r + 2] + buf[r + 6]
                a3 = a[3] + buf[r + 3] + buf[r + 7]
                return (a0, a1, a2, a3)

            a = lax.fori_loop(0, n8, blk, (zero, zero, zero, zero))

            def tail(r, t):
                return t + buf[r]

            return lax.fori_loop(mid, rh, tail,
                                 a[0] + a[1] + a[2] + a[3])

        seg0 = jnp.int32(0)
        e = total
        for j in range(B):
            seg0 = seg0 + (scs[j] <= g0).astype(jnp.int32)
        for j in range(B - 1, -1, -1):
            e = jnp.where(scs[j] > g0, scs[j], e)

        @pl.when(e >= g1)
        def _():
            res[seg0] = res[seg0] + accum(base + g0, base + g1)

        @pl.when(e < g1)
        def _():
            for b in range(B):
                s_b = scs[b - 1] if b else jnp.int32(0)
                e_b = scs[b]
                lo = jnp.minimum(jnp.maximum(s_b, g0), g1)
                hi = jnp.minimum(jnp.maximum(e_b, g0), g1)
                res[b, :] = res[b, :] + accum(base + lo, base + hi)

        @pl.when(c + NBUF < nch)
        def _():
            dcopy(c + NBUF, slot).start()

        return carry

    lax.fori_loop(0, nch, chunk_step, jnp.int32(0))

    for b in range(B):
        n_b = scs[b] - (scs[b - 1] if b else jnp.int32(0))
        res[b, :] = res[b, :] / n_b.astype(jnp.float32)

    pltpu.sync_copy(res, out_hbm.at[:, pl.ds(colbase, COLS)])


def kernel(x, lengths):
    f = functools.partial(
        pl.kernel,
        mesh=plsc.VectorSubcoreMesh(core_axis_name="c", subcore_axis_name="s"),
        out_type=jax.ShapeDtypeStruct((B, D), jnp.float32),
        scratch_types=[
            pltpu.VMEM((B,), jnp.int32),
            pltpu.VMEM((NBUF * CH, L), jnp.float32),
            pltpu.VMEM((B, L), jnp.float32),
            pltpu.SemaphoreType.DMA((NBUF,)),
        ],
        compiler_params=pltpu.CompilerParams(use_tc_tiling_on_sc=False),
    )(_body)
    cum = jnp.cumsum(lengths)
    return f(x, cum)
